# Initial kernel scaffold; baseline (speedup 1.0000x reference)
import jax
import jax.numpy as jnp
from jax.experimental import pallas as pl


def kernel(fingerprint, basis, W, b, base_cantor):
    raise NotImplementedError("write your pallas kernel here")


# speedup vs baseline: 2.7211x; 2.7211x over previous
import functools

import jax
import jax.numpy as jnp
from jax import lax
from jax.experimental import pallas as pl
from jax.experimental.pallas import tpu as pltpu
from jax.experimental.pallas import tpu_sc as plsc

_P = 262144
_D = 64
_NB = 32
_NW = 32
_PW = _P // _NW
_PPW = _PW // 2
_CHP = 128
_NCH = _PPW // _CHP


def _sc_lookup(scale_vec, shift_vec, cant_e, cant_o, basis2):
    mesh = plsc.VectorSubcoreMesh(core_axis_name="c", subcore_axis_name="s")

    @functools.partial(
        pl.kernel,
        out_type=jax.ShapeDtypeStruct((_P // 2, 2 * _D), jnp.float32),
        mesh=mesh,
        scratch_types=[
            pltpu.VMEM((16,), jnp.float32),
            pltpu.VMEM((16,), jnp.float32),
            pltpu.VMEM((_PPW,), jnp.float32),
            pltpu.VMEM((_PPW,), jnp.float32),
            pltpu.VMEM((_PPW,), jnp.int32),
            pltpu.VMEM((_CHP, 2 * _D), jnp.float32),
            pltpu.SemaphoreType.DMA,
        ],
    )
    def body(scale_hbm, shift_hbm, cant_e_hbm, cant_o_hbm, basis2_hbm, out_hbm,
             scale_v, shift_v, ce_v, co_v, pidx_v, rows_v, sem):
        wid = lax.axis_index("s") * 2 + lax.axis_index("c")
        pbase = wid * _PPW
        pltpu.sync_copy(scale_hbm, scale_v)
        pltpu.sync_copy(shift_hbm, shift_v)
        pltpu.sync_copy(cant_e_hbm.at[pl.ds(pbase, _PPW)], ce_v)
        pltpu.sync_copy(cant_o_hbm.at[pl.ds(pbase, _PPW)], co_v)
        s = scale_v[...]
        t = shift_v[...]

        def compute(i, carry):
            e = ce_v[pl.ds(i * 16, 16)]
            o = co_v[pl.ds(i * 16, 16)]
            ie = (e * s + t).astype(jnp.int32) & (_NB - 1)
            io = (o * s + t).astype(jnp.int32) & (_NB - 1)
            pidx_v[pl.ds(i * 16, 16)] = ie * _NB + io
            return carry

        lax.fori_loop(0, _PPW // 16, compute, 0)

        def emit(j, carry):
            off = j * _CHP
            pltpu.async_copy(
                basis2_hbm.at[pidx_v.at[pl.ds(off, _CHP)]], rows_v, sem
            ).wait()
            pltpu.sync_copy(rows_v, out_hbm.at[pl.ds(pbase + off, _CHP)])
            return carry

        lax.fori_loop(0, _NCH, emit, 0)

    return body(scale_vec, shift_vec, cant_e, cant_o, basis2)


def kernel(fingerprint, basis, W, b, base_cantor):
    params = W @ fingerprint + b
    scale = jax.nn.sigmoid(params[0]) * 2.0 + 0.5
    shift = jnp.sum(jnp.tanh(params[1:2]) * 512.0)
    scale_vec = jnp.full((16,), scale, jnp.float32)
    shift_vec = jnp.full((16,), shift, jnp.float32)
    left = jnp.broadcast_to(basis[:, None, :], (_NB, _NB, _D))
    right = jnp.broadcast_to(basis[None, :, :], (_NB, _NB, _D))
    basis2 = jnp.concatenate([left, right], axis=-1).reshape(_NB * _NB, 2 * _D)
    cant2 = base_cantor.reshape(_P // 2, 2)
    out2 = _sc_lookup(scale_vec, shift_vec, cant2[:, 0], cant2[:, 1], basis2)
    return out2.reshape(_P, _D)


# speedup vs baseline: 2.8222x; 1.0372x over previous
import functools

import jax
import jax.numpy as jnp
from jax import lax
from jax.experimental import pallas as pl
from jax.experimental.pallas import tpu as pltpu
from jax.experimental.pallas import tpu_sc as plsc

_P = 262144
_D = 64
_NB = 32
_NW = 32
_PW = _P // _NW
_PPW = _PW // 2
_CHP = 128
_NCH = _PPW // _CHP
_NBUF = 4


def _sc_lookup(scale_vec, shift_vec, cant_e, cant_o, basis2):
    mesh = plsc.VectorSubcoreMesh(core_axis_name="c", subcore_axis_name="s")

    @functools.partial(
        pl.kernel,
        out_type=jax.ShapeDtypeStruct((_P // 2, 2 * _D), jnp.float32),
        mesh=mesh,
        scratch_types=[
            pltpu.VMEM((16,), jnp.float32),
            pltpu.VMEM((16,), jnp.float32),
            pltpu.VMEM((_PPW,), jnp.float32),
            pltpu.VMEM((_PPW,), jnp.float32),
            pltpu.VMEM((_PPW,), jnp.int32),
            [pltpu.VMEM((_CHP, 2 * _D), jnp.float32) for _ in range(_NBUF)],
            [pltpu.SemaphoreType.DMA for _ in range(_NBUF)],
            [pltpu.SemaphoreType.DMA for _ in range(_NBUF)],
        ],
    )
    def body(scale_hbm, shift_hbm, cant_e_hbm, cant_o_hbm, basis2_hbm, out_hbm,
             scale_v, shift_v, ce_v, co_v, pidx_v, rows, gsem, osem):
        wid = lax.axis_index("s") * 2 + lax.axis_index("c")
        pbase = wid * _PPW
        pltpu.sync_copy(scale_hbm, scale_v)
        pltpu.sync_copy(shift_hbm, shift_v)
        pltpu.sync_copy(cant_e_hbm.at[pl.ds(pbase, _PPW)], ce_v)
        pltpu.sync_copy(cant_o_hbm.at[pl.ds(pbase, _PPW)], co_v)
        s = scale_v[...]
        t = shift_v[...]

        def compute(i, carry):
            e = ce_v[pl.ds(i * 16, 16)]
            o = co_v[pl.ds(i * 16, 16)]
            ie = (e * s + t).astype(jnp.int32) & (_NB - 1)
            io = (o * s + t).astype(jnp.int32) & (_NB - 1)
            pidx_v[pl.ds(i * 16, 16)] = ie * _NB + io
            return carry

        lax.fori_loop(0, _PPW // 16, compute, 0)

        def gather(k):
            return pltpu.async_copy(
                basis2_hbm.at[pidx_v.at[pl.ds(k * _CHP, _CHP)]],
                rows[k % _NBUF],
                gsem[k % _NBUF],
            )

        def outcopy(j):
            return pltpu.async_copy(
                rows[j % _NBUF],
                out_hbm.at[pl.ds(pbase + j * _CHP, _CHP)],
                osem[j % _NBUF],
            )

        gathers = {k: gather(k) for k in range(2)}
        outs = {}
        for j in range(_NCH):
            gathers[j].wait()
            outs[j] = outcopy(j)
            k = j + 2
            if k < _NCH:
                if k >= _NBUF:
                    outs[k - _NBUF].wait()
                gathers[k] = gather(k)
        for j in range(_NCH - _NBUF, _NCH):
            outs[j].wait()

    return body(scale_vec, shift_vec, cant_e, cant_o, basis2)


def kernel(fingerprint, basis, W, b, base_cantor):
    params = W @ fingerprint + b
    scale = jax.nn.sigmoid(params[0]) * 2.0 + 0.5
    shift = jnp.sum(jnp.tanh(params[1:2]) * 512.0)
    scale_vec = jnp.full((16,), scale, jnp.float32)
    shift_vec = jnp.full((16,), shift, jnp.float32)
    left = jnp.broadcast_to(basis[:, None, :], (_NB, _NB, _D))
    right = jnp.broadcast_to(basis[None, :, :], (_NB, _NB, _D))
    basis2 = jnp.concatenate([left, right], axis=-1).reshape(_NB * _NB, 2 * _D)
    cant2 = base_cantor.reshape(_P // 2, 2)
    out2 = _sc_lookup(scale_vec, shift_vec, cant2[:, 0], cant2[:, 1], basis2)
    return out2.reshape(_P, _D)


# speedup vs baseline: 4.4957x; 1.5930x over previous
import functools

import jax
import jax.numpy as jnp
from jax import lax
from jax.experimental import pallas as pl
from jax.experimental.pallas import tpu as pltpu
from jax.experimental.pallas import tpu_sc as plsc

_P = 262144
_D = 64
_NB = 32
_NW = 32
_PW = _P // _NW
_CH = 128
_NCH = _PW // _CH
_NBUF = 4


def _sc_lookup(scale_vec, shift_vec, base_cantor, basis):
    mesh = plsc.VectorSubcoreMesh(core_axis_name="c", subcore_axis_name="s")

    @functools.partial(
        pl.kernel,
        out_type=jax.ShapeDtypeStruct((_P, _D), jnp.float32),
        mesh=mesh,
        scratch_types=[
            pltpu.VMEM((16,), jnp.float32),
            pltpu.VMEM((16,), jnp.float32),
            pltpu.VMEM((_PW,), jnp.float32),
            pltpu.VMEM((_PW,), jnp.int32),
            pltpu.VMEM((_NB, _D), jnp.float32),
            [pltpu.VMEM((_CH, _D), jnp.float32) for _ in range(_NBUF)],
            [pltpu.SemaphoreType.DMA for _ in range(_NBUF)],
        ],
    )
    def body(scale_hbm, shift_hbm, cantor_hbm, basis_hbm, out_hbm,
             scale_v, shift_v, cant_v, idx_v, tab_v, bufs, osem):
        wid = lax.axis_index("s") * 2 + lax.axis_index("c")
        base = wid * _PW
        pltpu.sync_copy(scale_hbm, scale_v)
        pltpu.sync_copy(shift_hbm, shift_v)
        pltpu.sync_copy(basis_hbm, tab_v)
        pltpu.sync_copy(cantor_hbm.at[pl.ds(base, _PW)], cant_v)
        s = scale_v[...]
        t = shift_v[...]

        def compute(i, carry):
            c = cant_v[pl.ds(i * 16, 16)]
            idx_v[pl.ds(i * 16, 16)] = (c * s + t).astype(jnp.int32) & (_NB - 1)
            return carry

        lax.fori_loop(0, _PW // 16, compute, 0)

        def outer(jj, carry):
            for b in range(_NBUF):
                j = jj * _NBUF + b
                buf = bufs[b]

                @pl.when(jj > 0)
                def _wait():
                    pltpu.make_async_copy(
                        buf, out_hbm.at[pl.ds(base, _CH)], osem[b]
                    ).wait()

                def fill(g, carry2):
                    iv = idx_v[pl.ds((j * (_CH // 16) + g) * 16, 16)]
                    for l in range(16):
                        i = iv[l]
                        p = g * 16 + l
                        for k in range(_D // 16):
                            buf[p, pl.ds(k * 16, 16)] = tab_v[i, pl.ds(k * 16, 16)]
                    return carry2

                lax.fori_loop(0, _CH // 16, fill, 0)
                pltpu.async_copy(
                    buf, out_hbm.at[pl.ds(base + j * _CH, _CH)], osem[b]
                )
            return carry

        lax.fori_loop(0, _NCH // _NBUF, outer, 0)
        for b in range(_NBUF):
            pltpu.make_async_copy(
                bufs[b], out_hbm.at[pl.ds(base, _CH)], osem[b]
            ).wait()

    return body(scale_vec, shift_vec, base_cantor, basis)


def kernel(fingerprint, basis, W, b, base_cantor):
    params = W @ fingerprint + b
    scale = jax.nn.sigmoid(params[0]) * 2.0 + 0.5
    shift = jnp.sum(jnp.tanh(params[1:2]) * 512.0)
    scale_vec = jnp.full((16,), scale, jnp.float32)
    shift_vec = jnp.full((16,), shift, jnp.float32)
    return _sc_lookup(scale_vec, shift_vec, base_cantor, basis)


# speedup vs baseline: 18.6071x; 4.1389x over previous
import functools

import jax
import jax.numpy as jnp
from jax import lax
from jax.experimental import pallas as pl
from jax.experimental.pallas import tpu as pltpu
from jax.experimental.pallas import tpu_sc as plsc

_P = 262144
_D = 64
_NB = 32
_NW = 32
_PW = _P // _NW
_CH = 256
_NCH = _PW // _CH
_NPV = _CH // 16
_NBUF = 2

_DNUMS = lax.GatherDimensionNumbers(
    offset_dims=(), collapsed_slice_dims=(0,), start_index_map=(0,)
)


def _take16(vec, idx):
    return lax.gather(
        vec, idx[:, None], _DNUMS, slice_sizes=(1,),
        mode=lax.GatherScatterMode.PROMISE_IN_BOUNDS,
    )


def _sc_lookup(scale_vec, shift_vec, base_cantor, basis_t):
    mesh = plsc.VectorSubcoreMesh(core_axis_name="c", subcore_axis_name="s")

    @functools.partial(
        pl.kernel,
        out_type=jax.ShapeDtypeStruct((_D, _P), jnp.float32),
        mesh=mesh,
        scratch_types=[
            pltpu.VMEM((16,), jnp.float32),
            pltpu.VMEM((16,), jnp.float32),
            pltpu.VMEM((_PW,), jnp.float32),
            pltpu.VMEM((_D, _NB), jnp.float32),
            [pltpu.VMEM((_D, _CH), jnp.float32) for _ in range(_NBUF)],
            [pltpu.SemaphoreType.DMA for _ in range(_NBUF)],
        ],
    )
    def body(scale_hbm, shift_hbm, cantor_hbm, tabt_hbm, outt_hbm,
             scale_v, shift_v, cant_v, tabt_v, bufs, osem):
        wid = lax.axis_index("s") * 2 + lax.axis_index("c")
        base = wid * _PW
        pltpu.sync_copy(scale_hbm, scale_v)
        pltpu.sync_copy(shift_hbm, shift_v)
        pltpu.sync_copy(tabt_hbm, tabt_v)
        pltpu.sync_copy(cantor_hbm.at[pl.ds(base, _PW)], cant_v)
        s = scale_v[...]
        t = shift_v[...]

        def outer(jj, carry):
            for b in range(_NBUF):
                j = jj * _NBUF + b
                buf = bufs[b]

                @pl.when(jj > 0)
                def _wait():
                    pltpu.make_async_copy(
                        buf, outt_hbm.at[:, pl.ds(base, _CH)], osem[b]
                    ).wait()

                ilos = []
                msks = []
                for pv in range(_NPV):
                    c = cant_v[pl.ds(j * _CH + pv * 16, 16)]
                    iv = (c * s + t).astype(jnp.int32) & (_NB - 1)
                    ilos.append(iv & 15)
                    msks.append(iv < 16)

                def dloop(d, carry2):
                    lo = tabt_v[d, pl.ds(0, 16)]
                    hi = tabt_v[d, pl.ds(16, 16)]
                    for pv in range(_NPV):
                        g = jnp.where(
                            msks[pv], _take16(lo, ilos[pv]), _take16(hi, ilos[pv])
                        )
                        buf[d, pl.ds(pv * 16, 16)] = g
                    return carry2

                lax.fori_loop(0, _D, dloop, 0)
                pltpu.async_copy(
                    buf, outt_hbm.at[:, pl.ds(base + j * _CH, _CH)], osem[b]
                )
            return carry

        lax.fori_loop(0, _NCH // _NBUF, outer, 0)
        for b in range(_NBUF):
            pltpu.make_async_copy(
                bufs[b], outt_hbm.at[:, pl.ds(base, _CH)], osem[b]
            ).wait()

    return body(scale_vec, shift_vec, base_cantor, basis_t)


def kernel(fingerprint, basis, W, b, base_cantor):
    params = W @ fingerprint + b
    scale = jax.nn.sigmoid(params[0]) * 2.0 + 0.5
    shift = jnp.sum(jnp.tanh(params[1:2]) * 512.0)
    scale_vec = jnp.full((16,), scale, jnp.float32)
    shift_vec = jnp.full((16,), shift, jnp.float32)
    out_t = _sc_lookup(scale_vec, shift_vec, base_cantor, basis.T)
    return out_t.T
